# Initial kernel scaffold; baseline (speedup 1.0000x reference)
import functools

import jax
import jax.numpy as jnp
from jax import lax
from jax.experimental import pallas as pl
from jax.experimental.pallas import tpu as pltpu
from jax.experimental.pallas import tpu_sc as plsc

NUM_EMB = 1000000
DIM = 64
R = 8
SCALING = 2.0

NC = 2
NS = 16
NW = NC * NS
CHUNK = 128


def _sc_gather(xg, weight, lora_a, n_chunks):
    B = NW * n_chunks * CHUNK
    mesh = plsc.VectorSubcoreMesh(core_axis_name="c", subcore_axis_name="s",
                                  num_cores=NC)

    @functools.partial(
        pl.kernel,
        mesh=mesh,
        out_type=[
            jax.ShapeDtypeStruct((B, DIM), jnp.float32),
            jax.ShapeDtypeStruct((B, R), jnp.float32),
        ],
        scratch_types=[
            pltpu.VMEM((n_chunks, CHUNK), jnp.int32),
            pltpu.VMEM((CHUNK, DIM), jnp.float32),
            pltpu.VMEM((CHUNK, R), jnp.float32),
            pltpu.SemaphoreType.DMA,
            pltpu.SemaphoreType.DMA,
        ],
    )
    def gather_kernel(x_hbm, w_hbm, a_hbm, gw_hbm, ga_hbm,
                      idx_v, wbuf, abuf, sem_w, sem_a):
        cid = lax.axis_index("c")
        sid = lax.axis_index("s")
        wid = sid * NC + cid
        pltpu.sync_copy(x_hbm.at[wid], idx_v)

        def step(i, carry):
            cp_w = pltpu.async_copy(w_hbm.at[idx_v.at[i]], wbuf, sem_w)
            cp_a = pltpu.async_copy(a_hbm.at[idx_v.at[i]], abuf, sem_a)
            cp_w.wait()
            cp_a.wait()
            row0 = (wid * n_chunks + i) * CHUNK
            pltpu.sync_copy(wbuf, gw_hbm.at[pl.ds(row0, CHUNK)])
            pltpu.sync_copy(abuf, ga_hbm.at[pl.ds(row0, CHUNK)])
            return carry

        lax.fori_loop(0, n_chunks, step, 0)

    return gather_kernel(xg, weight, lora_a)


def _combine_body(gw_ref, ga_ref, b_ref, out_ref):
    out_ref[...] = gw_ref[...] + jnp.dot(
        ga_ref[...], b_ref[...], preferred_element_type=jnp.float32) * SCALING


def _tc_combine(gw, ga, lora_b, blk):
    B = gw.shape[0]
    grid = (B // blk,)
    return pl.pallas_call(
        _combine_body,
        grid=grid,
        in_specs=[
            pl.BlockSpec((blk, DIM), lambda i: (i, 0)),
            pl.BlockSpec((blk, R), lambda i: (i, 0)),
            pl.BlockSpec((R, DIM), lambda i: (0, 0)),
        ],
        out_specs=pl.BlockSpec((blk, DIM), lambda i: (i, 0)),
        out_shape=jax.ShapeDtypeStruct((B, DIM), jnp.float32),
    )(gw, ga, lora_b)


def kernel(x, weight, lora_a, lora_b):
    orig_shape = x.shape
    xf = x.reshape(-1)
    B = xf.shape[0]
    n_chunks = B // (NW * CHUNK)
    assert n_chunks * NW * CHUNK == B
    xg = xf.reshape(NW, n_chunks, CHUNK)
    gw, ga = _sc_gather(xg, weight, lora_a, n_chunks)
    out = _tc_combine(gw, ga, lora_b, blk=2048)
    return out.reshape(orig_shape + (DIM,))


# speedup vs baseline: 2.7483x; 2.7483x over previous
import functools

import jax
import jax.numpy as jnp
from jax import lax
from jax.experimental import pallas as pl
from jax.experimental.pallas import tpu as pltpu
from jax.experimental.pallas import tpu_sc as plsc

NUM_EMB = 1000000
DIM = 64
R = 8
SCALING = 2.0

NC = 2
NS = 16
NW = NC * NS
CHUNK = 128


def _sc_gather(xg, weight, lora_a, n_chunks):
    B = NW * n_chunks * CHUNK
    mesh = plsc.VectorSubcoreMesh(core_axis_name="c", subcore_axis_name="s",
                                  num_cores=NC)

    @functools.partial(
        pl.kernel,
        mesh=mesh,
        compiler_params=pltpu.CompilerParams(use_tc_tiling_on_sc=False),
        out_type=[
            jax.ShapeDtypeStruct((B, DIM), jnp.float32),
            jax.ShapeDtypeStruct((B, R), jnp.float32),
        ],
        scratch_types=[
            pltpu.VMEM((n_chunks, CHUNK), jnp.int32),
            pltpu.VMEM((CHUNK, DIM), jnp.float32),
            pltpu.VMEM((CHUNK, R), jnp.float32),
            pltpu.SemaphoreType.DMA,
            pltpu.SemaphoreType.DMA,
        ],
    )
    def gather_kernel(x_hbm, w_hbm, a_hbm, gw_hbm, ga_hbm,
                      idx_v, wbuf, abuf, sem_w, sem_a):
        cid = lax.axis_index("c")
        sid = lax.axis_index("s")
        wid = sid * NC + cid
        pltpu.sync_copy(x_hbm.at[wid], idx_v)

        def step(i, carry):
            cp_w = pltpu.async_copy(w_hbm.at[idx_v.at[i]], wbuf, sem_w)
            cp_a = pltpu.async_copy(a_hbm.at[idx_v.at[i]], abuf, sem_a)
            cp_w.wait()
            cp_a.wait()
            row0 = (wid * n_chunks + i) * CHUNK
            pltpu.sync_copy(wbuf, gw_hbm.at[pl.ds(row0, CHUNK)])
            pltpu.sync_copy(abuf, ga_hbm.at[pl.ds(row0, CHUNK)])
            return carry

        lax.fori_loop(0, n_chunks, step, 0)

    return gather_kernel(xg, weight, lora_a)


def _combine_body(gw_ref, ga_ref, b_ref, out_ref):
    out_ref[...] = gw_ref[...] + jnp.dot(
        ga_ref[...], b_ref[...], preferred_element_type=jnp.float32) * SCALING


def _tc_combine(gw, ga, lora_b, blk):
    B = gw.shape[0]
    grid = (B // blk,)
    return pl.pallas_call(
        _combine_body,
        grid=grid,
        in_specs=[
            pl.BlockSpec((blk, DIM), lambda i: (i, 0)),
            pl.BlockSpec((blk, R), lambda i: (i, 0)),
            pl.BlockSpec((R, DIM), lambda i: (0, 0)),
        ],
        out_specs=pl.BlockSpec((blk, DIM), lambda i: (i, 0)),
        out_shape=jax.ShapeDtypeStruct((B, DIM), jnp.float32),
    )(gw, ga, lora_b)


def kernel(x, weight, lora_a, lora_b):
    orig_shape = x.shape
    xf = x.reshape(-1)
    B = xf.shape[0]
    n_chunks = B // (NW * CHUNK)
    assert n_chunks * NW * CHUNK == B
    xg = xf.reshape(NW, n_chunks, CHUNK)
    gw, ga = _sc_gather(xg, weight, lora_a, n_chunks)
    out = _tc_combine(gw, ga, lora_b, blk=2048)
    return out.reshape(orig_shape + (DIM,))


# speedup vs baseline: 3.4711x; 1.2630x over previous
import functools

import jax
import jax.numpy as jnp
from jax import lax
from jax.experimental import pallas as pl
from jax.experimental.pallas import tpu as pltpu
from jax.experimental.pallas import tpu_sc as plsc

DIM = 64
R = 8
SCALING = 2.0

NC = 2
NS = 16
NW = NC * NS
IDXV = 128
NSTREAM = 4
CHUNK = IDXV * NSTREAM
UNROLL = 2
L = 16


def _sc_fused(xg, weight, lora_a, b2, n_chunks):
    B = NW * n_chunks * CHUNK
    mesh = plsc.VectorSubcoreMesh(core_axis_name="c", subcore_axis_name="s",
                                  num_cores=NC)

    @functools.partial(
        pl.kernel,
        mesh=mesh,
        compiler_params=pltpu.CompilerParams(use_tc_tiling_on_sc=False,
                                             needs_layout_passes=False),
        out_type=jax.ShapeDtypeStruct((B, DIM), jnp.float32),
        scratch_types=[
            pltpu.VMEM((n_chunks, NSTREAM, IDXV), jnp.int32),
            pltpu.VMEM((2 * CHUNK, DIM), jnp.float32),
            pltpu.VMEM((2 * CHUNK, R), jnp.float32),
            pltpu.VMEM((R, DIM), jnp.float32),
            pltpu.SemaphoreType.DMA,
            pltpu.SemaphoreType.DMA,
            pltpu.SemaphoreType.DMA,
        ],
    )
    def fused_kernel(x_hbm, w_hbm, a_hbm, b2_hbm, out_hbm,
                     idx_v, wbuf, abuf, bv, sem_w, sem_a, sem_b):
        cid = lax.axis_index("c")
        sid = lax.axis_index("s")
        wid = sid * NC + cid
        pltpu.sync_copy(x_hbm.at[wid], idx_v)
        pltpu.async_copy(b2_hbm, bv, sem_b).wait()

        bregs = [[bv[k, pl.ds(c * L, L)] for c in range(DIM // L)]
                 for k in range(R)]
        kf = [jnp.full((L,), k, jnp.int32) for k in range(R)]

        def fire(i, sb):
            for q in range(NSTREAM):
                pltpu.async_copy(w_hbm.at[idx_v.at[i, q]],
                                 wbuf.at[pl.ds(sb + q * IDXV, IDXV)], sem_w)
                pltpu.async_copy(a_hbm.at[idx_v.at[i, q]],
                                 abuf.at[pl.ds(sb + q * IDXV, IDXV)], sem_a)

        def drain(sb):
            for q in range(NSTREAM):
                pltpu.make_async_copy(
                    w_hbm.at[pl.ds(0, IDXV)],
                    wbuf.at[pl.ds(sb + q * IDXV, IDXV)], sem_w).wait()
                pltpu.make_async_copy(
                    a_hbm.at[pl.ds(0, IDXV)],
                    abuf.at[pl.ds(sb + q * IDXV, IDXV)], sem_a).wait()

        def compute(sb):
            def row_body(it, carry):
                for u in range(UNROLL):
                    r = sb + it * UNROLL + u
                    rfull = jnp.full((L,), r, jnp.int32)
                    accs = [wbuf[r, pl.ds(c * L, L)] for c in range(DIM // L)]
                    for k in range(R):
                        a_s = plsc.load_gather(abuf, [rfull, kf[k]])
                        for c in range(DIM // L):
                            accs[c] = accs[c] + a_s * bregs[k][c]
                    for c in range(DIM // L):
                        wbuf[r, pl.ds(c * L, L)] = accs[c]
                return carry

            lax.fori_loop(0, CHUNK // UNROLL, row_body, 0)

        def step(i, slot):
            sb = slot * CHUNK
            drain(sb)
            compute(sb)
            row0 = (wid * n_chunks + i) * CHUNK
            pltpu.sync_copy(wbuf.at[pl.ds(sb, CHUNK)],
                            out_hbm.at[pl.ds(row0, CHUNK)])

            @pl.when(i + 2 < n_chunks)
            def _():
                fire(i + 2, sb)

            return 1 - slot

        fire(0, 0)
        fire(1, CHUNK)
        lax.fori_loop(0, n_chunks, step, 0)

    return fused_kernel(xg, weight, lora_a, b2)


def kernel(x, weight, lora_a, lora_b):
    orig_shape = x.shape
    xf = x.reshape(-1)
    B = xf.shape[0]
    n_chunks = B // (NW * CHUNK)
    assert n_chunks * NW * CHUNK == B
    xg = xf.reshape(NW, n_chunks, NSTREAM, IDXV)
    b2 = lora_b * jnp.float32(SCALING)
    out = _sc_fused(xg, weight, lora_a, b2, n_chunks)
    return out.reshape(orig_shape + (DIM,))


# speedup vs baseline: 3.5174x; 1.0133x over previous
import functools

import jax
import jax.numpy as jnp
from jax import lax
from jax.experimental import pallas as pl
from jax.experimental.pallas import tpu as pltpu
from jax.experimental.pallas import tpu_sc as plsc

DIM = 64
R = 8
SCALING = 2.0

NC = 2
NS = 16
NW = NC * NS
S = 20
XR = 16
NBUF = 3
L = 16


def _sc_fused(x, weight, lora_a, b2):
    n_rows = x.shape[0]
    rows_pw = n_rows // NW
    n_chunks = rows_pw // XR
    mesh = plsc.VectorSubcoreMesh(core_axis_name="c", subcore_axis_name="s",
                                  num_cores=NC)

    @functools.partial(
        pl.kernel,
        mesh=mesh,
        compiler_params=pltpu.CompilerParams(use_tc_tiling_on_sc=False,
                                             needs_layout_passes=False),
        out_type=jax.ShapeDtypeStruct((n_rows, S, DIM), jnp.float32),
        scratch_types=[
            pltpu.VMEM((rows_pw, S), jnp.int32),
            pltpu.VMEM((NBUF, XR, S, DIM), jnp.float32),
            pltpu.VMEM((NBUF * XR * S, R), jnp.float32),
            pltpu.VMEM((R, DIM), jnp.float32),
            pltpu.SemaphoreType.DMA,
            pltpu.SemaphoreType.DMA,
            pltpu.SemaphoreType.DMA,
            pltpu.SemaphoreType.DMA,
        ],
    )
    def fused_kernel(x_hbm, w_hbm, a_hbm, b2_hbm, out_hbm,
                     idx_v, wbuf, abuf, bv, sem_w, sem_a, sem_b, sem_o):
        cid = lax.axis_index("c")
        sid = lax.axis_index("s")
        wid = sid * NC + cid
        x0 = wid * rows_pw
        pltpu.sync_copy(x_hbm.at[pl.ds(x0, rows_pw)], idx_v)
        pltpu.async_copy(b2_hbm, bv, sem_b).wait()

        bregs = [[bv[k, pl.ds(c * L, L)] for c in range(DIM // L)]
                 for k in range(R)]
        kf = [jnp.full((L,), k, jnp.int32) for k in range(R)]

        def g_copies(c, s):
            cps = []
            for j in range(XR):
                cps.append(pltpu.make_async_copy(
                    w_hbm.at[idx_v.at[c * XR + j]], wbuf.at[s, j], sem_w))
                cps.append(pltpu.make_async_copy(
                    a_hbm.at[idx_v.at[c * XR + j]],
                    abuf.at[pl.ds((s * XR + j) * S, S)], sem_a))
            return cps

        def o_copy(c, s):
            return pltpu.make_async_copy(
                wbuf.at[s], out_hbm.at[pl.ds(x0 + c * XR, XR)], sem_o)

        def compute(s):
            def row_body(xr, carry):
                for ss in range(S):
                    arow = jnp.full((L,), (s * XR + xr) * S + ss, jnp.int32)
                    accs = [wbuf[s, xr, ss, pl.ds(c * L, L)]
                            for c in range(DIM // L)]
                    for k in range(R):
                        a_s = plsc.load_gather(abuf, [arow, kf[k]])
                        for c in range(DIM // L):
                            accs[c] = accs[c] + a_s * bregs[k][c]
                    for c in range(DIM // L):
                        wbuf[s, xr, ss, pl.ds(c * L, L)] = accs[c]
                return carry

            lax.fori_loop(0, XR, row_body, 0)

        def step(c, s1, s2, s3):
            for cp in g_copies(c, s1):
                cp.wait()
            compute(s1)
            o_copy(c, s1).start()

            @pl.when(c + 2 < n_chunks)
            def _():
                @pl.when(c >= 1)
                def _():
                    o_copy(c - 1, s3).wait()
                for cp in g_copies(c + 2, s3):
                    cp.start()

        for cp in g_copies(0, 0):
            cp.start()
        for cp in g_copies(1, 1):
            cp.start()

        def trio(t, carry):
            for b in range(NBUF):
                step(t * NBUF + b, b, (b + 1) % NBUF, (b + 2) % NBUF)
            return carry

        lax.fori_loop(0, n_chunks // NBUF, trio, 0)
        step(jnp.int32(n_chunks - 2), 0, 1, 2)
        step(jnp.int32(n_chunks - 1), 1, 2, 0)
        for c, s in ((n_chunks - 3, 2), (n_chunks - 2, 0), (n_chunks - 1, 1)):
            o_copy(c, s).wait()

    return fused_kernel(x, weight, lora_a, b2)


def kernel(x, weight, lora_a, lora_b):
    b2 = lora_b * jnp.float32(SCALING)
    return _sc_fused(x, weight, lora_a, b2)
